# initial kernel scaffold (unmeasured)
import jax
import jax.numpy as jnp
from jax import lax
from jax.experimental import pallas as pl
from jax.experimental.pallas import tpu as pltpu

N_DEV = 4


def kernel(x, Win0, Wout0, Win1, Wout1, Win2, Wout2):
    B, D = x.shape
    rows = B // N_DEV

    def body(x_ref, win0, wout0, win1, wout1, win2, wout2, out_ref,
             ar_buf, rs_stage, rs_buf,
             ar_send_sems, ar_recv_sems, rs_send_sems, rs_recv_sems):
        my = lax.axis_index("i")

        barrier_sem = pltpu.get_barrier_semaphore()
        for k in range(1, N_DEV):
            pl.semaphore_signal(
                barrier_sem, inc=1,
                device_id=((my + k) % N_DEV,),
                device_id_type=pl.DeviceIdType.MESH,
            )
        pl.semaphore_wait(barrier_sem, N_DEV - 1)

        def mlp_partial(xb, win, wout):
            h = jnp.dot(xb, win[...].astype(jnp.bfloat16),
                        preferred_element_type=jnp.float32)
            h = jnp.maximum(h, 0.0).astype(jnp.bfloat16)
            return jnp.dot(h, wout[...].astype(jnp.bfloat16),
                           preferred_element_type=jnp.float32)

        xb = x_ref[...].astype(jnp.bfloat16)

        for r, (win, wout) in enumerate([(win0, wout0), (win1, wout1)]):
            p = mlp_partial(xb, win, wout)
            ar_buf[r, 0] = p.astype(jnp.bfloat16)
            rdmas = []
            for k in range(1, N_DEV):
                rdma = pltpu.make_async_remote_copy(
                    src_ref=ar_buf.at[r, 0],
                    dst_ref=ar_buf.at[r, k],
                    send_sem=ar_send_sems.at[r, k],
                    recv_sem=ar_recv_sems.at[r, k],
                    device_id=((my + k) % N_DEV,),
                    device_id_type=pl.DeviceIdType.MESH,
                )
                rdma.start()
                rdmas.append(rdma)
            for rdma in rdmas:
                rdma.wait_recv()
            total = p
            for k in range(1, N_DEV):
                total = total + ar_buf[r, k].astype(jnp.float32)
            for rdma in rdmas:
                rdma.wait_send()
            xb = total.astype(jnp.bfloat16)

        p2 = mlp_partial(xb, win2, wout2)
        rs_stage[...] = p2.astype(jnp.bfloat16)
        rs_rdmas = []
        for k in range(1, N_DEV):
            dest = (my + k) % N_DEV
            rdma = pltpu.make_async_remote_copy(
                src_ref=rs_stage.at[pl.ds(dest * rows, rows)],
                dst_ref=rs_buf.at[k],
                send_sem=rs_send_sems.at[k],
                recv_sem=rs_recv_sems.at[k],
                device_id=(dest,),
                device_id_type=pl.DeviceIdType.MESH,
            )
            rdma.start()
            rs_rdmas.append(rdma)
        for rdma in rs_rdmas:
            rdma.wait_recv()
        total = lax.dynamic_slice_in_dim(p2, my * rows, rows, axis=0)
        for k in range(1, N_DEV):
            total = total + rs_buf[k].astype(jnp.float32)
        for rdma in rs_rdmas:
            rdma.wait_send()
        out_ref[...] = total

    return pl.pallas_call(
        body,
        out_shape=jax.ShapeDtypeStruct((rows, D), jnp.float32),
        in_specs=[pl.BlockSpec(memory_space=pltpu.VMEM)] * 7,
        out_specs=pl.BlockSpec(memory_space=pltpu.VMEM),
        scratch_shapes=[
            pltpu.VMEM((2, N_DEV, B, D), jnp.bfloat16),
            pltpu.VMEM((B, D), jnp.bfloat16),
            pltpu.VMEM((N_DEV, rows, D), jnp.bfloat16),
            pltpu.SemaphoreType.DMA((2, N_DEV)),
            pltpu.SemaphoreType.DMA((2, N_DEV)),
            pltpu.SemaphoreType.DMA((N_DEV,)),
            pltpu.SemaphoreType.DMA((N_DEV,)),
        ],
        compiler_params=pltpu.CompilerParams(collective_id=0),
    )(x, Win0, Wout0, Win1, Wout1, Win2, Wout2)


# baseline (device time: 43723 ns/iter reference)
import jax
import jax.numpy as jnp
from jax import lax
from jax.experimental import pallas as pl
from jax.experimental.pallas import tpu as pltpu

N_DEV = 4


def kernel(x, Win0, Wout0, Win1, Wout1, Win2, Wout2):
    B, D = x.shape
    rows = B // N_DEV

    def body(x_ref, win0, wout0, win1, wout1, win2, wout2, out_ref,
             ar_buf, rs_stage, rs_buf,
             ar_send_sems, ar_recv_sems, rs_send_sems, rs_recv_sems):
        my = lax.axis_index("i")

        barrier_sem = pltpu.get_barrier_semaphore()
        for k in range(1, N_DEV):
            pl.semaphore_signal(
                barrier_sem, inc=1,
                device_id=((my + k) % N_DEV,),
                device_id_type=pl.DeviceIdType.MESH,
            )
        pl.semaphore_wait(barrier_sem, N_DEV - 1)

        def mlp_partial(xb, win, wout):
            h = jnp.dot(xb, win[...], preferred_element_type=jnp.float32)
            h = jnp.maximum(h, 0.0).astype(jnp.bfloat16)
            return jnp.dot(h, wout[...], preferred_element_type=jnp.float32)

        xb = x_ref[...].astype(jnp.bfloat16)

        for r, (win, wout) in enumerate([(win0, wout0), (win1, wout1)]):
            p = mlp_partial(xb, win, wout)
            ar_buf[r, 0] = p.astype(jnp.bfloat16)
            rdmas = []
            for k in range(1, N_DEV):
                rdma = pltpu.make_async_remote_copy(
                    src_ref=ar_buf.at[r, 0],
                    dst_ref=ar_buf.at[r, k],
                    send_sem=ar_send_sems.at[r, k],
                    recv_sem=ar_recv_sems.at[r, k],
                    device_id=((my + k) % N_DEV,),
                    device_id_type=pl.DeviceIdType.MESH,
                )
                rdma.start()
                rdmas.append(rdma)
            for rdma in rdmas:
                rdma.wait_recv()
            total = p
            for k in range(1, N_DEV):
                total = total + ar_buf[r, k].astype(jnp.float32)
            for rdma in rdmas:
                rdma.wait_send()
            xb = total.astype(jnp.bfloat16)

        p2 = mlp_partial(xb, win2, wout2)
        rs_stage[...] = p2.astype(jnp.bfloat16)
        rs_rdmas = []
        for k in range(1, N_DEV):
            dest = (my + k) % N_DEV
            rdma = pltpu.make_async_remote_copy(
                src_ref=rs_stage.at[pl.ds(dest * rows, rows)],
                dst_ref=rs_buf.at[k],
                send_sem=rs_send_sems.at[k],
                recv_sem=rs_recv_sems.at[k],
                device_id=(dest,),
                device_id_type=pl.DeviceIdType.MESH,
            )
            rdma.start()
            rs_rdmas.append(rdma)
        for rdma in rs_rdmas:
            rdma.wait_recv()
        total = rs_stage[pl.ds(my * rows, rows)].astype(jnp.float32)
        for k in range(1, N_DEV):
            total = total + rs_buf[k].astype(jnp.float32)
        for rdma in rs_rdmas:
            rdma.wait_send()
        out_ref[...] = total

    return pl.pallas_call(
        body,
        out_shape=jax.ShapeDtypeStruct((rows, D), jnp.float32),
        in_specs=[pl.BlockSpec(memory_space=pltpu.VMEM)] * 7,
        out_specs=pl.BlockSpec(memory_space=pltpu.VMEM),
        scratch_shapes=[
            pltpu.VMEM((2, N_DEV, B, D), jnp.bfloat16),
            pltpu.VMEM((B, D), jnp.bfloat16),
            pltpu.VMEM((N_DEV, rows, D), jnp.bfloat16),
            pltpu.SemaphoreType.DMA((2, N_DEV)),
            pltpu.SemaphoreType.DMA((2, N_DEV)),
            pltpu.SemaphoreType.DMA((N_DEV,)),
            pltpu.SemaphoreType.DMA((N_DEV,)),
        ],
        compiler_params=pltpu.CompilerParams(collective_id=0),
    )(
        x,
        Win0.astype(jnp.bfloat16),
        Wout0.astype(jnp.bfloat16),
        Win1.astype(jnp.bfloat16),
        Wout1.astype(jnp.bfloat16),
        Win2.astype(jnp.bfloat16),
        Wout2.astype(jnp.bfloat16),
    )


# device time: 31089 ns/iter; 1.4064x vs baseline; 1.4064x over previous
import jax
import jax.numpy as jnp
from jax import lax
from jax.experimental import pallas as pl
from jax.experimental.pallas import tpu as pltpu

N_DEV = 4


def kernel(x, Win0, Wout0, Win1, Wout1, Win2, Wout2):
    B, D = x.shape
    H = Win0.shape[1]
    rows = B // N_DEV

    def body(x_ref, win0, wout0, win1, wout1, win2, wout2, out_ref,
             winbuf, woutbuf, ar_buf, rs_stage, rs_buf,
             win_sem, wout_sem,
             ar_send_sems, ar_recv_sems, rs_send_sems, rs_recv_sems):
        my = lax.axis_index("i")
        wins = [win0, win1, win2]
        wouts = [wout0, wout1, wout2]
        win_dma = [pltpu.make_async_copy(wins[r], winbuf, win_sem)
                   for r in range(3)]
        wout_dma = [pltpu.make_async_copy(wouts[r], woutbuf, wout_sem)
                    for r in range(3)]

        win_dma[0].start()
        wout_dma[0].start()

        barrier_sem = pltpu.get_barrier_semaphore()
        for k in range(1, N_DEV):
            pl.semaphore_signal(
                barrier_sem, inc=1,
                device_id=((my + k) % N_DEV,),
                device_id_type=pl.DeviceIdType.MESH,
            )
        pl.semaphore_wait(barrier_sem, N_DEV - 1)

        xb = x_ref[...].astype(jnp.bfloat16)

        for r in range(3):
            win_dma[r].wait()
            h = jnp.dot(xb, winbuf[...].astype(jnp.bfloat16),
                        preferred_element_type=jnp.float32)
            h = jnp.maximum(h, 0.0).astype(jnp.bfloat16)
            wout_dma[r].wait()
            p = jnp.dot(h, woutbuf[...].astype(jnp.bfloat16),
                        preferred_element_type=jnp.float32)
            if r < 2:
                win_dma[r + 1].start()
                wout_dma[r + 1].start()

            if r < 2:
                ar_buf[r, 0] = p.astype(jnp.bfloat16)
                rdmas = []
                for k in range(1, N_DEV):
                    rdma = pltpu.make_async_remote_copy(
                        src_ref=ar_buf.at[r, 0],
                        dst_ref=ar_buf.at[r, k],
                        send_sem=ar_send_sems.at[r, k],
                        recv_sem=ar_recv_sems.at[r, k],
                        device_id=((my + k) % N_DEV,),
                        device_id_type=pl.DeviceIdType.MESH,
                    )
                    rdma.start()
                    rdmas.append(rdma)
                for rdma in rdmas:
                    rdma.wait_recv()
                total = p
                for k in range(1, N_DEV):
                    total = total + ar_buf[r, k].astype(jnp.float32)
                for rdma in rdmas:
                    rdma.wait_send()
                xb = total.astype(jnp.bfloat16)
            else:
                rs_stage[...] = p.astype(jnp.bfloat16)
                rs_rdmas = []
                for k in range(1, N_DEV):
                    dest = (my + k) % N_DEV
                    rdma = pltpu.make_async_remote_copy(
                        src_ref=rs_stage.at[pl.ds(dest * rows, rows)],
                        dst_ref=rs_buf.at[k],
                        send_sem=rs_send_sems.at[k],
                        recv_sem=rs_recv_sems.at[k],
                        device_id=(dest,),
                        device_id_type=pl.DeviceIdType.MESH,
                    )
                    rdma.start()
                    rs_rdmas.append(rdma)
                for rdma in rs_rdmas:
                    rdma.wait_recv()
                total = rs_stage[pl.ds(my * rows, rows)].astype(jnp.float32)
                for k in range(1, N_DEV):
                    total = total + rs_buf[k].astype(jnp.float32)
                for rdma in rs_rdmas:
                    rdma.wait_send()
                out_ref[...] = total

    return pl.pallas_call(
        body,
        out_shape=jax.ShapeDtypeStruct((rows, D), jnp.float32),
        in_specs=[pl.BlockSpec(memory_space=pltpu.VMEM)]
        + [pl.BlockSpec(memory_space=pl.ANY)] * 6,
        out_specs=pl.BlockSpec(memory_space=pltpu.VMEM),
        scratch_shapes=[
            pltpu.VMEM((D, H), jnp.float32),
            pltpu.VMEM((H, D), jnp.float32),
            pltpu.VMEM((2, N_DEV, B, D), jnp.bfloat16),
            pltpu.VMEM((B, D), jnp.bfloat16),
            pltpu.VMEM((N_DEV, rows, D), jnp.bfloat16),
            pltpu.SemaphoreType.DMA,
            pltpu.SemaphoreType.DMA,
            pltpu.SemaphoreType.DMA((2, N_DEV)),
            pltpu.SemaphoreType.DMA((2, N_DEV)),
            pltpu.SemaphoreType.DMA((N_DEV,)),
            pltpu.SemaphoreType.DMA((N_DEV,)),
        ],
        compiler_params=pltpu.CompilerParams(collective_id=0),
    )(x, Win0, Wout0, Win1, Wout1, Win2, Wout2)
